# baseline (speedup 1.0000x reference)
import functools

import jax
import jax.numpy as jnp
from jax import lax
from jax.experimental import pallas as pl
from jax.experimental.pallas import tpu as pltpu
from jax.experimental.pallas import tpu_sc as plsc

B, L = 4096, 50
DIM = 32
LANES = 16

NC, NS = 2, 16
NW = NC * NS
RPW = B // NW
ROWS_PER_CHUNK = 2
G = ROWS_PER_CHUNK * L
CHUNKS = RPW // ROWS_PER_CHUNK
LPAD = 64


def _permute(vec, idx):
    return lax.gather(
        vec, idx[:, None],
        dimension_numbers=lax.GatherDimensionNumbers(
            offset_dims=(), collapsed_slice_dims=(0,), start_index_map=(0,)),
        slice_sizes=(1,),
        mode=lax.GatherScatterMode.PROMISE_IN_BOUNDS)


def _bcast(vec, j):
    return _permute(vec, jnp.full((LANES,), j, jnp.int32))


def _allsum(vec):
    idx = jnp.arange(LANES, dtype=jnp.int32)
    for sh in (1, 2, 4, 8):
        vec = vec + _permute(vec, idx ^ sh)
    return vec


def _make_kernel():
    mesh = plsc.VectorSubcoreMesh(core_axis_name="c", subcore_axis_name="s")

    @functools.partial(
        pl.kernel,
        mesh=mesh,
        out_type=jax.ShapeDtypeStruct((B, DIM), jnp.float32),
        compiler_params=pltpu.CompilerParams(use_tc_tiling_on_sc=False),
        scratch_types=[
            pltpu.VMEM((CHUNKS + 1, G), jnp.int32),
            pltpu.VMEM((RPW, LPAD), jnp.float32),
            pltpu.VMEM((G, DIM), jnp.float32),
            pltpu.VMEM((G, DIM), jnp.float32),
            pltpu.VMEM((RPW, DIM), jnp.float32),
            pltpu.SemaphoreType.DMA,
            pltpu.SemaphoreType.DMA,
        ],
    )
    def emb(idx_hbm, w_hbm, table_hbm, out_hbm,
            idx_v, w_v, buf_a, buf_b, out_v, sem_a, sem_b):
        wid = lax.axis_index("s") * NC + lax.axis_index("c")
        pltpu.sync_copy(idx_hbm.at[wid], idx_v)
        pltpu.sync_copy(w_hbm.at[wid], w_v)

        def start(jj, buf, sem):
            pltpu.async_copy(table_hbm.at[idx_v.at[jj]], buf, sem)

        def wait(jj, buf, sem):
            pltpu.make_async_copy(table_hbm.at[idx_v.at[jj]], buf, sem).wait()

        def compute(buf, jj):
            for r in range(ROWS_PER_CHUNK):
                b = ROWS_PER_CHUNK * jj + r
                wblk = [w_v[b, pl.ds(k * LANES, LANES)] for k in range(4)]
                dv = wblk[0] + wblk[1] + wblk[2] + wblk[3]
                inv = 1.0 / _allsum(dv)
                a0 = jnp.zeros((LANES,), jnp.float32)
                a1 = jnp.zeros((LANES,), jnp.float32)
                for l in range(L):
                    w = _bcast(wblk[l // LANES], l % LANES)
                    pos = r * L + l
                    a0 = a0 + w * buf[pos, pl.ds(0, LANES)]
                    a1 = a1 + w * buf[pos, pl.ds(LANES, LANES)]
                out_v[b, pl.ds(0, LANES)] = a0 * inv
                out_v[b, pl.ds(LANES, LANES)] = a1 * inv

        start(0, buf_a, sem_a)

        def body(j2, carry):
            jj = 2 * j2
            wait(jj, buf_a, sem_a)
            start(jj + 1, buf_b, sem_b)
            compute(buf_a, jj)
            wait(jj + 1, buf_b, sem_b)
            start(jj + 2, buf_a, sem_a)
            compute(buf_b, jj + 1)
            return carry

        lax.fori_loop(0, CHUNKS // 2, body, 0)
        wait(CHUNKS, buf_a, sem_a)
        pltpu.sync_copy(out_v, out_hbm.at[pl.ds(wid * RPW, RPW)])

    return emb


_emb_kernel = _make_kernel()


def kernel(x_id, x_weight, word_vector):
    idx = x_id.astype(jnp.int32).reshape(NW, CHUNKS, G)
    idx = jnp.concatenate([idx, idx[:, :1]], axis=1)
    w = jnp.pad(x_weight.astype(jnp.float32), ((0, 0), (0, LPAD - L)))
    w = w.reshape(NW, RPW, LPAD)
    return _emb_kernel(idx, w, word_vector.astype(jnp.float32))
